# baseline (speedup 1.0000x reference)
import functools

import jax
import jax.numpy as jnp
from jax.experimental import pallas as pl
from jax.experimental.pallas import tpu as pltpu

_B, _C, _H, _W = 256, 3, 224, 224
_E, _N = 16, 1000
_HW = _H * _W
_HW_SUB = _HW // 128
_NQ = 4
_QROWS = _B // _NQ
_BLK_B = 8
_NSTEPS = _QROWS // _BLK_B


def _moe_body(x0_ref, x1_ref, x2_ref, x3_ref, Wg_ref, bg_ref, We_t_ref,
              be_ref, out_ref, w_ref, pooled_acc):
    i = pl.program_id(0)
    for j, xr in enumerate((x0_ref, x1_ref, x2_ref, x3_ref)):
        s = jnp.sum(xr[...], axis=(2, 3)) * (1.0 / _HW)
        pooled_acc[pl.ds(j * _QROWS + i * _BLK_B, _BLK_B), :] = s

    @pl.when(i == _NSTEPS - 1)
    def _finalize():
        pooled = pooled_acc[...]
        logits = (jnp.dot(pooled, Wg_ref[...],
                          preferred_element_type=jnp.float32) + bg_ref[...])
        weights = jax.nn.softmax(logits, axis=1)
        w_ref[...] = weights
        m = jnp.max(weights, axis=1, keepdims=True)
        lane = jax.lax.broadcasted_iota(jnp.int32, (_B, _E), 1)
        eidx = jnp.min(jnp.where(weights == m, lane, _E), axis=1,
                       keepdims=True)
        onehot = (lane == eidx).astype(jnp.float32)
        acc = jnp.dot(onehot, be_ref[...],
                      preferred_element_type=jnp.float32)
        for c in range(_C):
            mp = onehot * pooled[:, c:c + 1]
            acc = acc + jnp.dot(mp, We_t_ref[c],
                                preferred_element_type=jnp.float32)
        out_ref[...] = acc


def kernel(x, Wg, bg, We, be):
    We_t = We.transpose(1, 0, 2)
    bg2 = bg.reshape(1, _E)

    def xspec(j):
        return pl.BlockSpec((_BLK_B, _C, _H, _W),
                            lambda i, j=j: (j * _NSTEPS + i, 0, 0, 0))

    out, weights = pl.pallas_call(
        _moe_body,
        grid=(_NSTEPS,),
        in_specs=[xspec(0), xspec(1), xspec(2), xspec(3),
                  pl.BlockSpec((_C, _E), lambda i: (0, 0)),
                  pl.BlockSpec((1, _E), lambda i: (0, 0)),
                  pl.BlockSpec((_C, _E, _N), lambda i: (0, 0, 0)),
                  pl.BlockSpec((_E, _N), lambda i: (0, 0))],
        out_specs=[
            pl.BlockSpec((_B, _N), lambda i: (0, 0)),
            pl.BlockSpec((_B, _E), lambda i: (0, 0)),
        ],
        out_shape=[
            jax.ShapeDtypeStruct((_B, _N), jnp.float32),
            jax.ShapeDtypeStruct((_B, _E), jnp.float32),
        ],
        scratch_shapes=[pltpu.VMEM((_B, _C), jnp.float32)],
    )(x, x, x, x, Wg, bg2, We_t, be)
    return (out, weights)


# speedup vs baseline: 4.0761x; 4.0761x over previous
import functools

import jax
import jax.numpy as jnp
from jax import lax
from jax.experimental import pallas as pl
from jax.experimental.pallas import tpu as pltpu

_B, _C, _H, _W = 256, 3, 224, 224
_E, _N = 16, 1000
_HW = _H * _W
_BLK_H = 16
_NSTEPS = _H // _BLK_H


def _moe_body(xt_ref, Wg_ref, bg_ref, We_t_ref, be_ref, out_ref, w_ref,
              acc_ref):
    i = pl.program_id(0)
    s = jnp.sum(xt_ref[...], axis=(1, 2))

    @pl.when(i == 0)
    def _init():
        acc_ref[...] = s

    @pl.when(i > 0)
    def _accum():
        acc_ref[...] += s

    @pl.when(i == _NSTEPS - 1)
    def _finalize():
        pooled_t = acc_ref[...] * (1.0 / _HW)
        logits = lax.dot_general(pooled_t, Wg_ref[...],
                                 (((0,), (0,)), ((), ())),
                                 preferred_element_type=jnp.float32)
        logits = logits + bg_ref[...]
        weights = jax.nn.softmax(logits, axis=1)
        w_ref[...] = weights
        ec = lax.broadcasted_iota(jnp.int32, (_C, _C), 0)
        eye = (ec == ec.T).astype(jnp.float32)
        pooled = lax.dot_general(pooled_t, eye, (((0,), (0,)), ((), ())),
                                 preferred_element_type=jnp.float32)
        m = jnp.max(weights, axis=1, keepdims=True)
        lane = lax.broadcasted_iota(jnp.int32, (_B, _E), 1)
        eidx = jnp.min(jnp.where(weights == m, lane, _E), axis=1,
                       keepdims=True)
        onehot = (lane == eidx).astype(jnp.float32)
        acc = jnp.dot(onehot, be_ref[...],
                      preferred_element_type=jnp.float32)
        for c in range(_C):
            mp = onehot * pooled[:, c:c + 1]
            acc = acc + jnp.dot(mp, We_t_ref[c],
                                preferred_element_type=jnp.float32)
        out_ref[...] = acc


def kernel(x, Wg, bg, We, be):
    xt = jnp.transpose(x, (1, 2, 3, 0))
    We_t = We.transpose(1, 0, 2)
    bg2 = bg.reshape(1, _E)
    out, weights = pl.pallas_call(
        _moe_body,
        grid=(_NSTEPS,),
        in_specs=[
            pl.BlockSpec((_C, _BLK_H, _W, _B), lambda i: (0, i, 0, 0)),
            pl.BlockSpec((_C, _E), lambda i: (0, 0)),
            pl.BlockSpec((1, _E), lambda i: (0, 0)),
            pl.BlockSpec((_C, _E, _N), lambda i: (0, 0, 0)),
            pl.BlockSpec((_E, _N), lambda i: (0, 0)),
        ],
        out_specs=[
            pl.BlockSpec((_B, _N), lambda i: (0, 0)),
            pl.BlockSpec((_B, _E), lambda i: (0, 0)),
        ],
        out_shape=[
            jax.ShapeDtypeStruct((_B, _N), jnp.float32),
            jax.ShapeDtypeStruct((_B, _E), jnp.float32),
        ],
        scratch_shapes=[pltpu.VMEM((_C, _B), jnp.float32)],
    )(xt, Wg, bg2, We_t, be)
    return (out, weights)
